# Initial kernel scaffold; baseline (speedup 1.0000x reference)
import functools

import jax
import jax.numpy as jnp
from jax.experimental import pallas as pl


_BE = 2000
_BN = 2000


def _msg_body(gx_ref, ea_ref, scale_ref, wbe_ref, bbe_ref, out_ref):
    ee = jnp.dot(ea_ref[...], wbe_ref[...], preferred_element_type=jnp.float32)
    v = gx_ref[...] + ee + bbe_ref[...]
    g = jax.nn.gelu(v, approximate=False)
    out_ref[...] = g * scale_ref[...]


def _final_body(msum_ref, x_ref, dsi_ref, ddi_ref, wlin_ref, blin_ref, out_ref):
    sl = jax.nn.gelu(x_ref[...], approximate=False) * (dsi_ref[...] * ddi_ref[...])
    t = msum_ref[...] * ddi_ref[...] + sl
    out_ref[...] = (
        jnp.dot(t, wlin_ref[...].T, preferred_element_type=jnp.float32)
        + blin_ref[...]
    )


def _tc_msg(gx, edge_attr, scale, W_be, b_be):
    E, D = gx.shape
    K = edge_attr.shape[1]
    grid = (E // _BE,)
    return pl.pallas_call(
        _msg_body,
        grid=grid,
        in_specs=[
            pl.BlockSpec((_BE, D), lambda i: (i, 0)),
            pl.BlockSpec((_BE, K), lambda i: (i, 0)),
            pl.BlockSpec((_BE, 1), lambda i: (i, 0)),
            pl.BlockSpec((K, D), lambda i: (0, 0)),
            pl.BlockSpec((1, D), lambda i: (0, 0)),
        ],
        out_specs=pl.BlockSpec((_BE, D), lambda i: (i, 0)),
        out_shape=jax.ShapeDtypeStruct((E, D), jnp.float32),
    )(gx, edge_attr, scale, W_be, b_be)


def _tc_final(msum, x, dsi, ddi, W_lin, b_lin):
    N, D = x.shape
    grid = (N // _BN,)
    return pl.pallas_call(
        _final_body,
        grid=grid,
        in_specs=[
            pl.BlockSpec((_BN, D), lambda i: (i, 0)),
            pl.BlockSpec((_BN, D), lambda i: (i, 0)),
            pl.BlockSpec((_BN, 1), lambda i: (i, 0)),
            pl.BlockSpec((_BN, 1), lambda i: (i, 0)),
            pl.BlockSpec((D, D), lambda i: (0, 0)),
            pl.BlockSpec((1, D), lambda i: (0, 0)),
        ],
        out_specs=pl.BlockSpec((_BN, D), lambda i: (i, 0)),
        out_shape=jax.ShapeDtypeStruct((N, D), jnp.float32),
    )(msum, x, dsi, ddi, W_lin, b_lin)


def kernel(x, edge_index, edge_attr, edge_weight, W_be, b_be, W_lin, b_lin):
    N, D = x.shape
    E = edge_index.shape[1]
    row = edge_index[0].astype(jnp.int32)
    col = edge_index[1].astype(jnp.int32)

    deg_src = jnp.ones((N,), jnp.float32).at[row].add(1.0)
    deg_dst = jnp.ones((N,), jnp.float32).at[col].add(1.0)
    dsi = jax.lax.rsqrt(deg_src)
    ddi = jax.lax.rsqrt(deg_dst)

    gx = x[row]
    scale = (dsi[row] * edge_weight[:, 0])[:, None]

    msg = _tc_msg(gx, edge_attr, scale, W_be, b_be)

    msum = jnp.zeros((N, D), jnp.float32).at[col].add(msg)

    return _tc_final(msum, x, dsi[:, None], ddi[:, None], W_lin, b_lin)


# speedup vs baseline: 1.6718x; 1.6718x over previous
import functools

import jax
import jax.numpy as jnp
from jax.experimental import pallas as pl


_BE = 2000
_BN = 2000


def _gelu_exact(v):
    return 0.5 * v * (1.0 + jax.lax.erf(v * 0.7071067811865476))


def _msg_body(gx_ref, ea_ref, scale_ref, wbe_ref, bbe_ref, out_ref):
    ee = jnp.dot(ea_ref[...], wbe_ref[...], preferred_element_type=jnp.float32)
    v = gx_ref[...] + ee + bbe_ref[...]
    out_ref[...] = _gelu_exact(v) * scale_ref[...]


def _final_body(msum_ref, x_ref, dsi_ref, ddi_ref, wlin_ref, blin_ref, out_ref):
    sl = _gelu_exact(x_ref[...]) * (dsi_ref[...] * ddi_ref[...])
    t = msum_ref[...] * ddi_ref[...] + sl
    out_ref[...] = (
        jnp.dot(t, wlin_ref[...].T, preferred_element_type=jnp.float32)
        + blin_ref[...]
    )


def _tc_msg(gx, edge_attr, scale, W_be, b_be):
    E, D = gx.shape
    K = edge_attr.shape[1]
    grid = (E // _BE,)
    return pl.pallas_call(
        _msg_body,
        grid=grid,
        in_specs=[
            pl.BlockSpec((_BE, D), lambda i: (i, 0)),
            pl.BlockSpec((_BE, K), lambda i: (i, 0)),
            pl.BlockSpec((_BE, 1), lambda i: (i, 0)),
            pl.BlockSpec((K, D), lambda i: (0, 0)),
            pl.BlockSpec((1, D), lambda i: (0, 0)),
        ],
        out_specs=pl.BlockSpec((_BE, D), lambda i: (i, 0)),
        out_shape=jax.ShapeDtypeStruct((E, D), jnp.float32),
    )(gx, edge_attr, scale, W_be, b_be)


def _tc_final(msum, x, dsi, ddi, W_lin, b_lin):
    N, D = x.shape
    grid = (N // _BN,)
    return pl.pallas_call(
        _final_body,
        grid=grid,
        in_specs=[
            pl.BlockSpec((_BN, D), lambda i: (i, 0)),
            pl.BlockSpec((_BN, D), lambda i: (i, 0)),
            pl.BlockSpec((_BN, 1), lambda i: (i, 0)),
            pl.BlockSpec((_BN, 1), lambda i: (i, 0)),
            pl.BlockSpec((D, D), lambda i: (0, 0)),
            pl.BlockSpec((1, D), lambda i: (0, 0)),
        ],
        out_specs=pl.BlockSpec((_BN, D), lambda i: (i, 0)),
        out_shape=jax.ShapeDtypeStruct((N, D), jnp.float32),
    )(msum, x, dsi, ddi, W_lin, b_lin)


def kernel(x, edge_index, edge_attr, edge_weight, W_be, b_be, W_lin, b_lin):
    N, D = x.shape
    E = edge_index.shape[1]
    row = edge_index[0].astype(jnp.int32)
    col = edge_index[1].astype(jnp.int32)

    deg_src = jnp.ones((N,), jnp.float32).at[row].add(1.0)
    deg_dst = jnp.ones((N,), jnp.float32).at[col].add(1.0)
    dsi = jax.lax.rsqrt(deg_src)
    ddi = jax.lax.rsqrt(deg_dst)

    gx = x[row]
    scale = (dsi[row] * edge_weight[:, 0])[:, None]

    msg = _tc_msg(gx, edge_attr, scale, W_be, b_be[None, :])

    msum = jnp.zeros((N, D), jnp.float32).at[col].add(msg)

    return _tc_final(msum, x, dsi[:, None], ddi[:, None], W_lin, b_lin[None, :])


# speedup vs baseline: 6.5282x; 3.9049x over previous
import functools

import jax
import jax.numpy as jnp
from jax import lax
from jax.experimental import pallas as pl
from jax.experimental.pallas import tpu as pltpu
from jax.experimental.pallas import tpu_sc as plsc

_N = 10000
_E = 320000
_D = 128
_K = 16
_NP = 10240
_EP = 327680
_NC = 2
_NS = 16
_NW = _NC * _NS
_CH = 128
_NCHUNK = _EP // _NW // _CH
_RPT = _NP // _NS

_BE = 2048
_BN = 2048
_NB = _NP // _BN


def _gelu_exact(v):
    return 0.5 * v * (1.0 + jax.lax.erf(v * 0.7071067811865476))


def _sc_mesh():
    return plsc.VectorSubcoreMesh(core_axis_name="c", subcore_axis_name="s")


def _worker():
    c = lax.axis_index("c")
    s = lax.axis_index("s")
    return c, s, s * _NC + c


def _deg_body(row2_hbm, col2_hbm, ones_hbm, z16_hbm, dsrc_hbm, ddst_hbm,
              idxr, idxc, ones_v, ds_sh, dd_sh, sem_a, sem_b):
    c, s, w = _worker()
    pltpu.sync_copy(ones_hbm, ones_v)
    pltpu.sync_copy(z16_hbm, ds_sh.at[pl.ds(s * _RPT, _RPT)])
    pltpu.sync_copy(z16_hbm, dd_sh.at[pl.ds(s * _RPT, _RPT)])
    pltpu.sync_copy(row2_hbm.at[pl.ds(w * _NCHUNK, _NCHUNK)], idxr)
    pltpu.sync_copy(col2_hbm.at[pl.ds(w * _NCHUNK, _NCHUNK)], idxc)
    plsc.subcore_barrier()
    grp = 8
    for g in range(0, _NCHUNK, grp):
        descs = []
        for j in range(g, g + grp):
            descs.append(
                pltpu.async_copy(ones_v, ds_sh.at[idxr.at[j]], sem_a, add=True))
            descs.append(
                pltpu.async_copy(ones_v, dd_sh.at[idxc.at[j]], sem_b, add=True))
        for d in descs:
            d.wait()
    plsc.subcore_barrier()
    off = c * _NP + s * _RPT
    pltpu.sync_copy(ds_sh.at[pl.ds(s * _RPT, _RPT)], dsrc_hbm.at[pl.ds(off, _RPT)])
    pltpu.sync_copy(dd_sh.at[pl.ds(s * _RPT, _RPT)], ddst_hbm.at[pl.ds(off, _RPT)])


def _sc_degrees(row2, col2, ones_h, z16):
    f = pl.kernel(
        _deg_body,
        out_type=(
            jax.ShapeDtypeStruct((_NC * _NP, 16), jnp.float32),
            jax.ShapeDtypeStruct((_NC * _NP, 16), jnp.float32),
        ),
        mesh=_sc_mesh(),
        scratch_types=[
            pltpu.VMEM((_NCHUNK, _CH), jnp.int32),
            pltpu.VMEM((_NCHUNK, _CH), jnp.int32),
            pltpu.VMEM((_CH, 16), jnp.float32),
            pltpu.VMEM_SHARED((_NP, 16), jnp.float32),
            pltpu.VMEM_SHARED((_NP, 16), jnp.float32),
            pltpu.SemaphoreType.DMA,
            pltpu.SemaphoreType.DMA,
        ],
    )
    return f(row2, col2, ones_h, z16)


def _build_body(ds0, ds1, dd0, dd1, dsi_ref, ddi_ref):
    dsrc = ds0[...] + ds1[...] + 1.0
    ddst = dd0[...] + dd1[...] + 1.0
    dsi_ref[...] = lax.rsqrt(dsrc)
    ddi_ref[...] = lax.rsqrt(ddst)


def _tc_build(dsrc, ddst):
    return pl.pallas_call(
        _build_body,
        grid=(_NB,),
        in_specs=[
            pl.BlockSpec((_BN, 16), lambda i: (i, 0)),
            pl.BlockSpec((_BN, 16), lambda i: (i + _NB, 0)),
            pl.BlockSpec((_BN, 16), lambda i: (i, 0)),
            pl.BlockSpec((_BN, 16), lambda i: (i + _NB, 0)),
        ],
        out_specs=[
            pl.BlockSpec((_BN, 16), lambda i: (i, 0)),
            pl.BlockSpec((_BN, 16), lambda i: (i, 0)),
        ],
        out_shape=[
            jax.ShapeDtypeStruct((_NP, 16), jnp.float32),
            jax.ShapeDtypeStruct((_NP, 16), jnp.float32),
        ],
    )(dsrc, dsrc, ddst, ddst)


def _gather_body(xpad_hbm, dsi_hbm, row2_hbm, gx_hbm, gd_hbm,
                 idxb, gx0, gx1, dsi_v, norm_v, sg0, sg1, sw):
    c, s, w = _worker()
    base = w * _NCHUNK
    pltpu.sync_copy(row2_hbm.at[pl.ds(base, _NCHUNK)], idxb)
    pltpu.sync_copy(dsi_hbm, dsi_v)
    gxb = [gx0, gx1]
    sg = [sg0, sg1]
    wdescs = [None, None]
    prev = None
    for j in range(_NCHUNK):
        p = j & 1
        if wdescs[p] is not None:
            wdescs[p].wait()
            wdescs[p] = None
        dg = pltpu.async_copy(xpad_hbm.at[idxb.at[j]], gxb[p], sg[p])
        for k in range(_CH // 16):
            idx16 = idxb[j, pl.ds(k * 16, 16)]
            norm_v[pl.ds(j * _CH + k * 16, 16)] = plsc.load_gather(
                dsi_v, [idx16])
        if prev is not None:
            pj, pdg = prev
            pdg.wait()
            q = pj & 1
            r0 = (base + pj) * _CH
            wdescs[q] = pltpu.async_copy(
                gxb[q], gx_hbm.at[pl.ds(r0, _CH)], sw)
        prev = (j, dg)
    pj, pdg = prev
    pdg.wait()
    q = pj & 1
    r0 = (base + pj) * _CH
    pltpu.sync_copy(gxb[q], gx_hbm.at[pl.ds(r0, _CH)])
    if wdescs[1 - q] is not None:
        wdescs[1 - q].wait()
    pltpu.sync_copy(norm_v, gd_hbm.at[pl.ds(w * _NCHUNK * _CH, _NCHUNK * _CH)])


def _sc_gather(xpad, dsi_flat, row2):
    f = pl.kernel(
        _gather_body,
        out_type=(
            jax.ShapeDtypeStruct((_EP, _D), jnp.float32),
            jax.ShapeDtypeStruct((_EP,), jnp.float32),
        ),
        mesh=_sc_mesh(),
        scratch_types=[
            pltpu.VMEM((_NCHUNK, _CH), jnp.int32),
            pltpu.VMEM((_CH, _D), jnp.float32),
            pltpu.VMEM((_CH, _D), jnp.float32),
            pltpu.VMEM((_NP,), jnp.float32),
            pltpu.VMEM((_NCHUNK * _CH,), jnp.float32),
            pltpu.SemaphoreType.DMA,
            pltpu.SemaphoreType.DMA,
            pltpu.SemaphoreType.DMA,
        ],
        compiler_params=pltpu.CompilerParams(needs_layout_passes=False),
    )
    return f(xpad, dsi_flat, row2)


def _msg_body(gx_ref, ea_ref, gd_ref, ew_ref, wbe_ref, bbe_ref, out_ref):
    ee = jnp.dot(ea_ref[...], wbe_ref[...], preferred_element_type=jnp.float32)
    v = gx_ref[...] + ee + bbe_ref[...]
    scale = gd_ref[...] * ew_ref[...]
    out_ref[...] = _gelu_exact(v) * scale


def _tc_msg(gx, eap, gd, ewp, W_be, b_be):
    return pl.pallas_call(
        _msg_body,
        grid=(_EP // _BE,),
        in_specs=[
            pl.BlockSpec((_BE, _D), lambda i: (i, 0)),
            pl.BlockSpec((_BE, _K), lambda i: (i, 0)),
            pl.BlockSpec((_BE, 1), lambda i: (i, 0)),
            pl.BlockSpec((_BE, 1), lambda i: (i, 0)),
            pl.BlockSpec((_K, _D), lambda i: (0, 0)),
            pl.BlockSpec((1, _D), lambda i: (0, 0)),
        ],
        out_specs=pl.BlockSpec((_BE, _D), lambda i: (i, 0)),
        out_shape=jax.ShapeDtypeStruct((_EP, _D), jnp.float32),
    )(gx, eap, gd, ewp, W_be, b_be)


def _scatter_body(msg_hbm, col2_hbm, z128_hbm, msum_hbm,
                  idxb, mb0, mb1, acc_sh, sl0, sl1, sa0, sa1):
    c, s, w = _worker()
    base = w * _NCHUNK
    pltpu.sync_copy(col2_hbm.at[pl.ds(base, _NCHUNK)], idxb)
    pltpu.sync_copy(z128_hbm, acc_sh.at[pl.ds(s * _RPT, _RPT)])
    plsc.subcore_barrier()
    mb = [mb0, mb1]
    sl = [sl0, sl1]
    sa = [sa0, sa1]
    adescs = [None, None]
    prev = None
    for j in range(_NCHUNK):
        p = j & 1
        if adescs[p] is not None:
            adescs[p].wait()
            adescs[p] = None
        dl = pltpu.async_copy(
            msg_hbm.at[pl.ds((base + j) * _CH, _CH)], mb[p], sl[p])
        if prev is not None:
            pj, pdl = prev
            pdl.wait()
            q = pj & 1
            adescs[q] = pltpu.async_copy(
                mb[q], acc_sh.at[idxb.at[pj]], sa[q], add=True)
        prev = (j, dl)
    pj, pdl = prev
    pdl.wait()
    q = pj & 1
    adescs[q] = pltpu.async_copy(mb[q], acc_sh.at[idxb.at[pj]], sa[q], add=True)
    for d in adescs:
        if d is not None:
            d.wait()
    plsc.subcore_barrier()
    pltpu.sync_copy(acc_sh.at[pl.ds(s * _RPT, _RPT)],
                    msum_hbm.at[pl.ds(c * _NP + s * _RPT, _RPT)])


def _sc_scatter(msg, col2, z128):
    f = pl.kernel(
        _scatter_body,
        out_type=jax.ShapeDtypeStruct((_NC * _NP, _D), jnp.float32),
        mesh=_sc_mesh(),
        scratch_types=[
            pltpu.VMEM((_NCHUNK, _CH), jnp.int32),
            pltpu.VMEM((_CH, _D), jnp.float32),
            pltpu.VMEM((_CH, _D), jnp.float32),
            pltpu.VMEM_SHARED((_NP, _D), jnp.float32),
            pltpu.SemaphoreType.DMA,
            pltpu.SemaphoreType.DMA,
            pltpu.SemaphoreType.DMA,
            pltpu.SemaphoreType.DMA,
        ],
    )
    return f(msg, col2, z128)


def _final_body(m0, m1, x_ref, dsi_ref, ddi_ref, wlin_ref, blin_ref, out_ref):
    ddi = ddi_ref[:, :1]
    dsi = dsi_ref[:, :1]
    sl = _gelu_exact(x_ref[...]) * (dsi * ddi)
    t = (m0[...] + m1[...]) * ddi + sl
    out_ref[...] = (
        jnp.dot(t, wlin_ref[...].T, preferred_element_type=jnp.float32)
        + blin_ref[...]
    )


def _tc_final(msum, xpad, dsi16, ddi16, W_lin, b_lin):
    return pl.pallas_call(
        _final_body,
        grid=(_NB,),
        in_specs=[
            pl.BlockSpec((_BN, _D), lambda i: (i, 0)),
            pl.BlockSpec((_BN, _D), lambda i: (i + _NB, 0)),
            pl.BlockSpec((_BN, _D), lambda i: (i, 0)),
            pl.BlockSpec((_BN, 16), lambda i: (i, 0)),
            pl.BlockSpec((_BN, 16), lambda i: (i, 0)),
            pl.BlockSpec((_D, _D), lambda i: (0, 0)),
            pl.BlockSpec((1, _D), lambda i: (0, 0)),
        ],
        out_specs=pl.BlockSpec((_BN, _D), lambda i: (i, 0)),
        out_shape=jax.ShapeDtypeStruct((_NP, _D), jnp.float32),
    )(msum, msum, xpad, dsi16, ddi16, W_lin, b_lin)


def kernel(x, edge_index, edge_attr, edge_weight, W_be, b_be, W_lin, b_lin):
    ei = edge_index.astype(jnp.int32)
    pad_e = _EP - _E
    pad_idx = jnp.full((pad_e,), _N, jnp.int32)
    row2 = jnp.concatenate([ei[0], pad_idx]).reshape(_NW * _NCHUNK, _CH)
    col2 = jnp.concatenate([ei[1], pad_idx]).reshape(_NW * _NCHUNK, _CH)

    xpad = jnp.concatenate([x, jnp.zeros((_NP - _N, _D), jnp.float32)], axis=0)
    eap = jnp.concatenate(
        [edge_attr, jnp.zeros((pad_e, _K), jnp.float32)], axis=0)
    ewp = jnp.concatenate(
        [edge_weight, jnp.zeros((pad_e, 1), jnp.float32)], axis=0)

    ones_h = jnp.ones((_CH, 16), jnp.float32)
    z16 = jnp.zeros((_RPT, 16), jnp.float32)
    z128 = jnp.zeros((_RPT, _D), jnp.float32)

    dsrc, ddst = _sc_degrees(row2, col2, ones_h, z16)
    dsi16, ddi16 = _tc_build(dsrc, ddst)
    gx, gd = _sc_gather(xpad, dsi16[:, 0], row2)
    msg = _tc_msg(gx, eap, gd[:, None], ewp, W_be, b_be[None, :])
    msum = _sc_scatter(msg, col2, z128)
    out = _tc_final(msum, xpad, dsi16, ddi16, W_lin, b_lin[None, :])
    return out[:_N]
